# Initial kernel scaffold; baseline (speedup 1.0000x reference)
import jax
import jax.numpy as jnp
from jax.experimental import pallas as pl


def kernel(x, edge_index, edge_attr, batch, params):
    raise NotImplementedError("write your pallas kernel here")


# speedup vs baseline: 1.0074x; 1.0074x over previous
import functools
import math

import jax
import jax.numpy as jnp
from jax.experimental import pallas as pl

_EPS = 1e-7
_BM = 512


def _mm_kern(a_ref, w_ref, o_ref, *, mode):
    z = jnp.dot(a_ref[...], w_ref[...], preferred_element_type=jnp.float32)
    if mode == "relu_eps":
        z = jnp.maximum(z, 0.0) + _EPS
    elif mode == "relu":
        z = jnp.maximum(z, 0.0)
    o_ref[...] = z


def _mm(a, w, mode="none"):
    m, k = a.shape
    n = w.shape[1]
    mp = ((m + _BM - 1) // _BM) * _BM
    if mp != m:
        a = jnp.pad(a, ((0, mp - m), (0, 0)))
    out = pl.pallas_call(
        functools.partial(_mm_kern, mode=mode),
        grid=(mp // _BM,),
        in_specs=[
            pl.BlockSpec((_BM, k), lambda i: (i, 0)),
            pl.BlockSpec((k, n), lambda i: (0, 0)),
        ],
        out_specs=pl.BlockSpec((_BM, n), lambda i: (i, 0)),
        out_shape=jax.ShapeDtypeStruct((mp, n), jnp.float32),
    )(a, w)
    return out[:m]


def _batchnorm(h, g, b):
    m = h.mean(axis=0)
    v = h.var(axis=0)
    return (h - m) / jnp.sqrt(v + 1e-5) * g + b


def _softmax_agg(msg, dst, n, emask):
    neg = jnp.where(emask[:, None] > 0, msg, -jnp.inf)
    mx = jax.ops.segment_max(neg, dst, num_segments=n)
    mx = jnp.where(jnp.isfinite(mx), mx, 0.0)
    ex = jnp.exp(msg - mx[dst]) * emask[:, None]
    den = jax.ops.segment_sum(ex, dst, num_segments=n)
    alpha = ex / (den[dst] + 1e-16)
    return jax.ops.segment_sum(msg * alpha, dst, num_segments=n)


def _genconv_relu(x, src, dst, emask, p, name):
    n = x.shape[0]
    msg_tab = _mm(x, p[name + "_Wsrc"], mode="relu_eps")
    msg = msg_tab[src]
    out = _softmax_agg(msg, dst, n, emask) + _mm(x, p[name + "_Wdst"])
    z = _mm(out, p[name + "_W1"])
    h1 = jax.nn.relu(_batchnorm(z, p[name + "_g1"], p[name + "_b1"]))
    return _mm(h1, p[name + "_W2"], mode="relu")


def _sagpool(x, src, dst, emask, batch, p, name, ratio):
    n = x.shape[0]
    agg = jax.ops.segment_sum(x[src] * emask[:, None], dst, num_segments=n)
    score = jnp.tanh(
        (agg @ p[name + "_Wrel"] + p[name + "_brel"] + x @ p[name + "_Wroot"]).reshape(-1)
    )
    kk = int(math.ceil(ratio * n))
    topv, perm = jax.lax.top_k(score, kk)
    xk = x[perm] * topv[:, None]
    newidx = (
        jnp.full((n,), -1, dtype=src.dtype)
        .at[perm]
        .set(jnp.arange(kk, dtype=src.dtype))
    )
    s2 = newidx[src]
    d2 = newidx[dst]
    valid = (s2 >= 0) & (d2 >= 0) & (emask > 0)
    return (
        xk,
        jnp.where(valid, s2, 0),
        jnp.where(valid, d2, 0),
        valid.astype(x.dtype),
        batch[perm],
    )


def kernel(x, edge_index, edge_attr, batch, params):
    p = params
    src, dst = edge_index[0], edge_index[1]
    emask = jnp.ones((src.shape[0],), x.dtype)
    x = _genconv_relu(x, src, dst, emask, p, "c1")
    x, src, dst, emask, batch = _sagpool(x, src, dst, emask, batch, p, "p1", 0.5)
    x = _genconv_relu(x, src, dst, emask, p, "c2")
    x, src, dst, emask, batch = _sagpool(x, src, dst, emask, batch, p, "p2", 0.5)
    x = _genconv_relu(x, src, dst, emask, p, "c3")
    x, src, dst, emask, batch = _sagpool(x, src, dst, emask, batch, p, "p3", 0.5)
    s = jax.ops.segment_sum(x, batch, num_segments=1)
    cnt = jax.ops.segment_sum(jnp.ones((x.shape[0],), x.dtype), batch, num_segments=1)
    g = s / jnp.maximum(cnt, 1.0)[:, None]
    h = jax.nn.relu(g @ p["lin1_W"] + p["lin1_b"])
    h = jax.nn.relu(h @ p["lin2_W"] + p["lin2_b"])
    return jax.nn.log_softmax(h @ p["lin3_W"] + p["lin3_b"], axis=-1)
